# baseline (speedup 1.0000x reference)
import functools

import jax
import jax.numpy as jnp
from jax import lax
from jax.experimental import pallas as pl
from jax.experimental.pallas import tpu as pltpu
from jax.experimental.pallas import tpu_sc as plsc

N_STAY = 50000
N_DIAG = 10000
E = 300000
F_IN = 128
HID = 64
H = 2
D = 32
NC = 3
NL = 2

N_TILES = 16
IC = 256
CHUNKS_PER_TILE = 78
EP = N_TILES * CHUNKS_PER_TILE * IC

NDP_STAY = 50048
NDP_DIAG = 10240


def _row_split(ndp):
    rpt = ndp // N_TILES
    cw = 8
    for d in range(8, 137, 8):
        if rpt % d == 0:
            cw = d
    return rpt, cw


def _norm(m, den, g, b):
    bn = m.shape[0]
    dd = jnp.concatenate(
        [jnp.broadcast_to(den[:, 0:1], (bn, D)),
         jnp.broadcast_to(den[:, 1:2], (bn, D))], axis=-1)
    v = jnp.maximum(m / (dd + 1e-16), 0.0)
    mu = jnp.mean(v, axis=-1, keepdims=True)
    var = jnp.mean((v - mu) ** 2, axis=-1, keepdims=True)
    return (v - mu) * lax.rsqrt(var + 1e-5) * g + b


def _proj_outs(y, oz_ref, os_ref, od_ref):
    oz_ref[...] = y[:, :HID]
    os_ref[...] = y[:, HID:HID + 2]
    od_ref[...] = y[:, HID + 2:HID + 4]


def _fin_body(x_ref, w1_ref, b1_ref, w2_ref, b2_ref, oz_ref, os_ref, od_ref):
    h = jnp.maximum(
        jnp.dot(x_ref[...], w1_ref[...], preferred_element_type=jnp.float32)
        + b1_ref[...], 0.0)
    y = jnp.dot(h, w2_ref[...], preferred_element_type=jnp.float32) + b2_ref[...]
    _proj_outs(y, oz_ref, os_ref, od_ref)


def _fmid_body(m_ref, d_ref, g_ref, b_ref, w2_ref, b2_ref,
               oz_ref, os_ref, od_ref):
    h = _norm(m_ref[...], d_ref[...], g_ref[...], b_ref[...])
    y = jnp.dot(h, w2_ref[...], preferred_element_type=jnp.float32) + b2_ref[...]
    _proj_outs(y, oz_ref, os_ref, od_ref)


def _fout_body(m_ref, d_ref, g_ref, b_ref, w2_ref, b2_ref, o_ref):
    h = _norm(m_ref[...], d_ref[...], g_ref[...], b_ref[...])
    o_ref[...] = (jnp.dot(h, w2_ref[...], preferred_element_type=jnp.float32)
                  + b2_ref[...])


def _proj_out_specs(n, bn):
    return (
        (jax.ShapeDtypeStruct((n, HID), jnp.float32),
         jax.ShapeDtypeStruct((n, 2), jnp.float32),
         jax.ShapeDtypeStruct((n, 2), jnp.float32)),
        (pl.BlockSpec((bn, HID), lambda i: (i, 0)),
         pl.BlockSpec((bn, 2), lambda i: (i, 0)),
         pl.BlockSpec((bn, 2), lambda i: (i, 0))),
    )


def _fin(x, w1, b1, w2, b2, bn=1000):
    n, k = x.shape
    f = w2.shape[1]
    assert n % bn == 0
    out_shape, out_specs = _proj_out_specs(n, bn)
    return pl.pallas_call(
        _fin_body,
        out_shape=out_shape,
        grid=(n // bn,),
        in_specs=[
            pl.BlockSpec((bn, k), lambda i: (i, 0)),
            pl.BlockSpec((k, HID), lambda i: (0, 0)),
            pl.BlockSpec((1, HID), lambda i: (0, 0)),
            pl.BlockSpec((HID, f), lambda i: (0, 0)),
            pl.BlockSpec((1, f), lambda i: (0, 0)),
        ],
        out_specs=out_specs,
    )(x, w1, b1.reshape(1, HID), w2, b2.reshape(1, f))


def _norm_specs(bn, f):
    return [
        pl.BlockSpec((bn, HID), lambda i: (i, 0)),
        pl.BlockSpec((bn, H), lambda i: (i, 0)),
        pl.BlockSpec((1, HID), lambda i: (0, 0)),
        pl.BlockSpec((1, HID), lambda i: (0, 0)),
        pl.BlockSpec((HID, f), lambda i: (0, 0)),
        pl.BlockSpec((1, f), lambda i: (0, 0)),
    ]


def _fmid(msg, den, g, b, w2, b2, bn):
    n = msg.shape[0]
    f = w2.shape[1]
    assert n % bn == 0
    out_shape, out_specs = _proj_out_specs(n, bn)
    return pl.pallas_call(
        _fmid_body,
        out_shape=out_shape,
        grid=(n // bn,),
        in_specs=_norm_specs(bn, f),
        out_specs=out_specs,
    )(msg, den, g.reshape(1, HID), b.reshape(1, HID), w2, b2.reshape(1, f))


def _fout(msg, den, g, b, w2, b2, bn):
    n = msg.shape[0]
    f = w2.shape[1]
    assert n % bn == 0
    return pl.pallas_call(
        _fout_body,
        out_shape=jax.ShapeDtypeStruct((n, f), jnp.float32),
        grid=(n // bn,),
        in_specs=_norm_specs(bn, f),
        out_specs=pl.BlockSpec((bn, f), lambda i: (i, 0)),
    )(msg, den, g.reshape(1, HID), b.reshape(1, HID), w2, b2.reshape(1, f))


def _sc_phase(c, s, ns, ndp, rpt, cw,
              zflat, alsrc, aldst, srce, dste, zrows0, zden0,
              msg_out, den_out,
              eb_s, eb_d, gidxb, gdstb, alsb, aldb, exc,
              zrow, msgb, bounce, denb, semi, semg, semz, semd, accum, dena):
    nchunk = CHUNKS_PER_TILE
    tbase = s * (nchunk * IC)
    cns = c * ns
    cnd = c * ndp

    def issue_idx(i, b):
        off = tbase + jnp.minimum(i, nchunk - 1) * IC
        pltpu.async_copy(srce.at[pl.ds(off, IC)], eb_s.at[b], semi.at[b])
        pltpu.async_copy(dste.at[pl.ds(off, 128)], eb_d.at[b, 0], semi.at[b])
        pltpu.async_copy(dste.at[pl.ds(off + 128, 128)], eb_d.at[b, 1],
                         semi.at[b])

    def wait_idx(b):
        pltpu.make_async_copy(srce.at[pl.ds(0, IC)], eb_s.at[b],
                              semi.at[b]).wait()
        for j in range(2):
            pltpu.make_async_copy(dste.at[pl.ds(0, 128)], eb_d.at[b, j],
                                  semi.at[b]).wait()

    def build(b):
        for j in range(2):
            for h in range(8):
                sv = eb_s[b, pl.ds(j * 128 + h * 16, 16)]
                dv = eb_d[b, j, pl.ds(h * 16, 16)]
                gidxb[b, j, pl.ds(h * 16, 16)] = sv + cns
                gdstb[b, j, pl.ds(h * 16, 16)] = dv + cnd

    def issue_gathers(b):
        for j in range(2):
            pltpu.async_copy(alsrc.at[gidxb.at[b, j]],
                             alsb.at[b, pl.ds(j * 128, 128)], semg.at[b])
            pltpu.async_copy(aldst.at[gdstb.at[b, j]],
                             aldb.at[b, pl.ds(j * 128, 128)], semg.at[b])
            pltpu.async_copy(zflat.at[gidxb.at[b, j]],
                             zrow.at[b, pl.ds(j * 128, 128)], semz.at[b])

    def wait_al(b):
        for j in range(2):
            pltpu.make_async_copy(alsrc.at[gidxb.at[b, j]],
                                  alsb.at[b, pl.ds(j * 128, 128)],
                                  semg.at[b]).wait()
            pltpu.make_async_copy(aldst.at[gdstb.at[b, j]],
                                  aldb.at[b, pl.ds(j * 128, 128)],
                                  semg.at[b]).wait()

    def wait_z(b):
        for j in range(2):
            pltpu.make_async_copy(zflat.at[gidxb.at[b, j]],
                                  zrow.at[b, pl.ds(j * 128, 128)],
                                  semz.at[b]).wait()

    def compute_scatter(b):
        wait_al(b)
        exvals = []
        for g in range(IC // 16):
            av = alsb[b, pl.ds(g * 16, 16)] + aldb[b, pl.ds(g * 16, 16)]
            av = jnp.where(av >= 0, av, av * 0.2)
            ex = jnp.exp(av)
            exvals.append(ex)
            exc[pl.ds(g * 16, 16)] = ex
        wait_z(b)
        for e in range(IC):
            exs = exvals[e // 16][e % 16]
            lo, hi = plsc.unpack(plsc.bitcast(zrow[b, e, :], jnp.bfloat16),
                                 format=plsc.PackFormat.INTERLEAVED)
            msgb[e, pl.ds(0, 16)] = lo * exs
            msgb[e, pl.ds(16, 16)] = hi * exs
        for j in range(2):
            pltpu.async_copy(exc.at[pl.ds(j * 128, 128)],
                             dena.at[eb_d.at[b, j]], semd.at[b], add=True)
        for j in range(2):
            pltpu.sync_copy(msgb.at[pl.ds(j * 128, 128)],
                            accum.at[eb_d.at[b, j]], add=True)
        for j in range(2):
            pltpu.make_async_copy(exc.at[pl.ds(j * 128, 128)],
                                  dena.at[eb_d.at[b, j]], semd.at[b]).wait()

    issue_idx(0, 0)
    issue_idx(1, 1)

    pltpu.sync_copy(zrows0, bounce)
    pltpu.sync_copy(zden0, denb)
    base = s * rpt
    for k in range(rpt // cw):
        pltpu.sync_copy(bounce, accum.at[pl.ds(base + k * cw, cw)])
    pltpu.sync_copy(denb, dena.at[pl.ds(base, rpt)])
    plsc.subcore_barrier()

    wait_idx(0)
    build(0)
    issue_gathers(0)

    def body(kk, carry):
        t = kk * 2
        for b in range(2):
            i = t + b
            nb = (b + 1) % 2
            wait_idx(nb)
            build(nb)
            issue_gathers(nb)
            compute_scatter(b)
            issue_idx(i + 2, b)
        return carry

    lax.fori_loop(0, nchunk // 2, body, 0)

    wait_al(nchunk % 2)
    wait_z(nchunk % 2)
    wait_idx((nchunk + 1) % 2)

    plsc.subcore_barrier()

    for k in range(rpt // cw):
        r = base + k * cw
        pltpu.sync_copy(accum.at[pl.ds(r, cw)], bounce)
        pltpu.sync_copy(bounce, msg_out.at[pl.ds(r, cw), c])
    pltpu.sync_copy(dena.at[pl.ds(base, rpt)], denb)
    pltpu.sync_copy(denb, den_out.at[pl.ds(c * ndp + base, rpt)])


def _sc_conv_body(ns, ndp, rpt, cw,
                  zflat, alsrc, aldst, srce, dste, zrows0, zden0,
                  msg_out, den_out, *scratch):
    c = lax.axis_index("c")
    s = lax.axis_index("s")
    _sc_phase(c, s, ns, ndp, rpt, cw, zflat, alsrc, aldst, srce, dste,
              zrows0, zden0, msg_out, den_out, *scratch)


def _sc2_body(ns_s, ndp_s, rpt_s, cw_s, ns_d, ndp_d, rpt_d, cw_d,
              zf_d, als_d, ald_d, e1s, e1d,
              zf_s, als_s, ald_s, e2s, e2d, zrows0, zden0,
              msgs_out, dens_out, msgd_out, dend_out, *scratch):
    c = lax.axis_index("c")
    s = lax.axis_index("s")
    sc_front, bounce, denb, sc_tail = scratch[:9], scratch[9], scratch[10], scratch[11:]
    _sc_phase(c, s, ns_s, ndp_s, rpt_s, cw_s, zf_d, als_d, ald_d, e1s, e1d,
              zrows0.at[pl.ds(0, cw_s)], zden0.at[pl.ds(0, rpt_s)],
              msgs_out, dens_out,
              *sc_front, bounce.at[pl.ds(0, cw_s)], denb.at[pl.ds(0, rpt_s)],
              *sc_tail)
    plsc.subcore_barrier()
    _sc_phase(c, s, ns_d, ndp_d, rpt_d, cw_d, zf_s, als_s, ald_s, e2s, e2d,
              zrows0.at[pl.ds(0, cw_d)], zden0.at[pl.ds(0, rpt_d)],
              msgd_out, dend_out,
              *sc_front, bounce.at[pl.ds(0, cw_d)], denb.at[pl.ds(0, rpt_d)],
              *sc_tail)


def _sc_scratch(cw, rpt, ndp):
    return [
        pltpu.VMEM((2, IC), jnp.int32),
        pltpu.VMEM((2, 2, 128), jnp.int32),
        pltpu.VMEM((2, 2, 128), jnp.int32),
        pltpu.VMEM((2, 2, 128), jnp.int32),
        pltpu.VMEM((2, IC), jnp.float32),
        pltpu.VMEM((2, IC), jnp.float32),
        pltpu.VMEM((IC,), jnp.float32),
        pltpu.VMEM((2, IC, 16), jnp.uint32),
        pltpu.VMEM((IC, D), jnp.float32),
        pltpu.VMEM((cw, D), jnp.float32),
        pltpu.VMEM((rpt,), jnp.float32),
        pltpu.SemaphoreType.DMA((2,)),
        pltpu.SemaphoreType.DMA((2,)),
        pltpu.SemaphoreType.DMA((2,)),
        pltpu.SemaphoreType.DMA((2,)),
        pltpu.VMEM_SHARED((ndp, D), jnp.float32),
        pltpu.VMEM_SHARED((ndp,), jnp.float32),
    ]


def _sc_conv2(zf_d, als_d, ald_d, e1s, e1d, zf_s, als_s, ald_s, e2s, e2d):
    rpt_s, cw_s = _row_split(NDP_STAY)
    rpt_d, cw_d = _row_split(NDP_DIAG)
    mesh = plsc.VectorSubcoreMesh(core_axis_name="c", subcore_axis_name="s",
                                  num_cores=2, num_subcores=N_TILES)
    fn = pl.kernel(
        functools.partial(_sc2_body, N_DIAG, NDP_STAY, rpt_s, cw_s,
                          N_STAY, NDP_DIAG, rpt_d, cw_d),
        out_type=(
            jax.ShapeDtypeStruct((NDP_STAY, 2, D), jnp.float32),
            jax.ShapeDtypeStruct((2 * NDP_STAY,), jnp.float32),
            jax.ShapeDtypeStruct((NDP_DIAG, 2, D), jnp.float32),
            jax.ShapeDtypeStruct((2 * NDP_DIAG,), jnp.float32),
        ),
        mesh=mesh,
        compiler_params=pltpu.CompilerParams(needs_layout_passes=False,
                                             use_tc_tiling_on_sc=False),
        scratch_types=_sc_scratch(max(cw_s, cw_d), max(rpt_s, rpt_d), NDP_STAY),
    )
    zrows0 = jnp.zeros((max(cw_s, cw_d), D), jnp.float32)
    zden0 = jnp.zeros((max(rpt_s, rpt_d),), jnp.float32)
    ms, ds_, md, dd = fn(zf_d, als_d, ald_d, e1s, e1d,
                         zf_s, als_s, ald_s, e2s, e2d, zrows0, zden0)
    return (ms.reshape(NDP_STAY, HID), ds_.reshape(2, NDP_STAY).T,
            md.reshape(NDP_DIAG, HID), dd.reshape(2, NDP_DIAG).T)


def _sc_conv(zflat, alsrc, aldst_p, src_p, dst_p, ns, ndp):
    rpt, cw = _row_split(ndp)
    mesh = plsc.VectorSubcoreMesh(core_axis_name="c", subcore_axis_name="s",
                                  num_cores=2, num_subcores=N_TILES)
    fn = pl.kernel(
        functools.partial(_sc_conv_body, ns, ndp, rpt, cw),
        out_type=(
            jax.ShapeDtypeStruct((ndp, 2, D), jnp.float32),
            jax.ShapeDtypeStruct((2 * ndp,), jnp.float32),
        ),
        mesh=mesh,
        compiler_params=pltpu.CompilerParams(needs_layout_passes=False,
                                             use_tc_tiling_on_sc=False),
        scratch_types=[
            pltpu.VMEM((2, IC), jnp.int32),
            pltpu.VMEM((2, 2, 128), jnp.int32),
            pltpu.VMEM((2, 2, 128), jnp.int32),
            pltpu.VMEM((2, 2, 128), jnp.int32),
            pltpu.VMEM((2, IC), jnp.float32),
            pltpu.VMEM((2, IC), jnp.float32),
            pltpu.VMEM((IC,), jnp.float32),
            pltpu.VMEM((2, IC, 16), jnp.uint32),
            pltpu.VMEM((IC, D), jnp.float32),
            pltpu.VMEM((cw, D), jnp.float32),
            pltpu.VMEM((rpt,), jnp.float32),
            pltpu.SemaphoreType.DMA((2,)),
            pltpu.SemaphoreType.DMA((2,)),
            pltpu.SemaphoreType.DMA((2,)),
            pltpu.SemaphoreType.DMA((2,)),
            pltpu.VMEM_SHARED((ndp, D), jnp.float32),
            pltpu.VMEM_SHARED((ndp,), jnp.float32),
        ],
    )
    zrows0 = jnp.zeros((cw, D), jnp.float32)
    zden0 = jnp.zeros((rpt,), jnp.float32)
    msg, den = fn(zflat, alsrc, aldst_p, src_p, dst_p, zrows0, zden0)
    return msg.reshape(ndp, HID), den.reshape(2, ndp).T


def _block_attn_mat(a):
    z = jnp.zeros((D, 1), jnp.float32)
    return jnp.block([[a[0][:, None], z], [z, a[1][:, None]]])


def _pad_al(al, ndp):
    n = al.shape[0]
    return jnp.concatenate(
        [al, jnp.zeros((ndp - n, 2), jnp.float32)], axis=0).T.reshape(-1)


def _hm(al):
    return al.T.reshape(-1)


def _zhm(z, n):
    zb = z.astype(jnp.bfloat16).reshape(n, H, 2, 16)
    zt = zb.transpose(1, 0, 3, 2)
    zu = jax.lax.bitcast_convert_type(zt, jnp.uint32)
    return zu.reshape(H * n, 16)


def _layer_weights(p, l):
    a_src_d2s = _block_attn_mat(p[f"l{l}_asrc_d2s"])
    a_dst_d2s = _block_attn_mat(p[f"l{l}_adst_d2s"])
    a_src_s2d = _block_attn_mat(p[f"l{l}_asrc_s2d"])
    a_dst_s2d = _block_attn_mat(p[f"l{l}_adst_s2d"])
    w_d, b_d = p[f"l{l}_proj_diag_W"], p[f"l{l}_proj_diag_b"]
    w_s, b_s = p[f"l{l}_proj_stay_W"], p[f"l{l}_proj_stay_b"]
    wcat_d = jnp.concatenate([w_d, w_d @ a_src_d2s, w_d @ a_dst_s2d], axis=1)
    bcat_d = jnp.concatenate([b_d, b_d @ a_src_d2s, b_d @ a_dst_s2d])
    wcat_s = jnp.concatenate([w_s, w_s @ a_src_s2d, w_s @ a_dst_d2s], axis=1)
    bcat_s = jnp.concatenate([b_s, b_s @ a_src_s2d, b_s @ a_dst_d2s])
    return wcat_d, bcat_d, wcat_s, bcat_s


def kernel(x_stay, x_diag, params, ei_d2s_src, ei_d2s_dst, ei_s2d_src, ei_s2d_dst):
    p = params
    pad = EP - E
    e1s = jnp.concatenate([ei_d2s_src, jnp.zeros((pad,), jnp.int32)])
    e1d = jnp.concatenate([ei_d2s_dst, jnp.full((pad,), N_STAY, jnp.int32)])
    e2s = jnp.concatenate([ei_s2d_src, jnp.zeros((pad,), jnp.int32)])
    e2d = jnp.concatenate([ei_s2d_dst, jnp.full((pad,), N_DIAG, jnp.int32)])

    wcat_d0, bcat_d0, wcat_s0, bcat_s0 = _layer_weights(p, 0)
    z_diag, alsrc_d2s, aldst_s2d = _fin(
        x_diag, p["in_diag_W"], p["in_diag_b"], wcat_d0, bcat_d0)
    z_stay, alsrc_s2d, aldst_d2s = _fin(
        x_stay, p["in_stay_W"], p["in_stay_b"], wcat_s0, bcat_s0)

    msg_s, den_s, msg_d, den_d = _sc_conv2(
        _zhm(z_diag, N_DIAG), _hm(alsrc_d2s), _pad_al(aldst_d2s, NDP_STAY),
        e1s, e1d,
        _zhm(z_stay, N_STAY), _hm(alsrc_s2d), _pad_al(aldst_s2d, NDP_DIAG),
        e2s, e2d)

    wcat_d1, bcat_d1, wcat_s1, bcat_s1 = _layer_weights(p, 1)
    z_diag1, alsrc_d2s1, _ = _fmid(
        msg_d, den_d, p["l0_ln_g"], p["l0_ln_b"], wcat_d1, bcat_d1, bn=1024)
    z_stay1, _, aldst_d2s1 = _fmid(
        msg_s, den_s, p["l0_ln_g"], p["l0_ln_b"], wcat_s1, bcat_s1, bn=1088)

    msg_s1, den_s1 = _sc_conv(_zhm(z_diag1, NDP_DIAG), _hm(alsrc_d2s1),
                              _hm(aldst_d2s1),
                              e1s, e1d, NDP_DIAG, NDP_STAY)

    out = _fout(msg_s1, den_s1, p["l1_ln_g"], p["l1_ln_b"],
                p["clf_W"], p["clf_b"], bn=1088)
    return out[:N_STAY]


# speedup vs baseline: 1.0542x; 1.0542x over previous
import functools

import jax
import jax.numpy as jnp
from jax import lax
from jax.experimental import pallas as pl
from jax.experimental.pallas import tpu as pltpu
from jax.experimental.pallas import tpu_sc as plsc

N_STAY = 50000
N_DIAG = 10000
E = 300000
F_IN = 128
HID = 64
H = 2
D = 32
NC = 3
NL = 2

N_TILES = 16
IC = 256
CHUNKS_PER_TILE = 78
EP = N_TILES * CHUNKS_PER_TILE * IC

NDP_STAY = 50048
NDP_DIAG = 10240


def _row_split(ndp):
    rpt = ndp // N_TILES
    cw = 8
    for d in range(8, 137, 8):
        if rpt % d == 0:
            cw = d
    return rpt, cw


def _norm(m, den, g, b):
    bn = m.shape[0]
    dd = jnp.concatenate(
        [jnp.broadcast_to(den[:, 0:1], (bn, D)),
         jnp.broadcast_to(den[:, 1:2], (bn, D))], axis=-1)
    v = jnp.maximum(m / (dd + 1e-16), 0.0)
    mu = jnp.mean(v, axis=-1, keepdims=True)
    var = jnp.mean((v - mu) ** 2, axis=-1, keepdims=True)
    return (v - mu) * lax.rsqrt(var + 1e-5) * g + b


def _proj_outs(y, oz_ref, os_ref, od_ref):
    oz_ref[...] = y[:, :HID]
    os_ref[...] = y[:, HID:HID + 2]
    od_ref[...] = y[:, HID + 2:HID + 4]


def _fin_body(x_ref, w1_ref, b1_ref, w2_ref, b2_ref, oz_ref, os_ref, od_ref):
    h = jnp.maximum(
        jnp.dot(x_ref[...], w1_ref[...], preferred_element_type=jnp.float32)
        + b1_ref[...], 0.0)
    y = jnp.dot(h, w2_ref[...], preferred_element_type=jnp.float32) + b2_ref[...]
    _proj_outs(y, oz_ref, os_ref, od_ref)


def _fmid_body(m_ref, d_ref, g_ref, b_ref, w2_ref, b2_ref,
               oz_ref, os_ref, od_ref):
    h = _norm(m_ref[...], d_ref[...], g_ref[...], b_ref[...])
    y = jnp.dot(h, w2_ref[...], preferred_element_type=jnp.float32) + b2_ref[...]
    _proj_outs(y, oz_ref, os_ref, od_ref)


def _fout_body(m_ref, d_ref, g_ref, b_ref, w2_ref, b2_ref, o_ref):
    h = _norm(m_ref[...], d_ref[...], g_ref[...], b_ref[...])
    o_ref[...] = (jnp.dot(h, w2_ref[...], preferred_element_type=jnp.float32)
                  + b2_ref[...])


def _proj_out_specs(n, bn):
    return (
        (jax.ShapeDtypeStruct((n, HID), jnp.float32),
         jax.ShapeDtypeStruct((n, 2), jnp.float32),
         jax.ShapeDtypeStruct((n, 2), jnp.float32)),
        (pl.BlockSpec((bn, HID), lambda i: (i, 0)),
         pl.BlockSpec((bn, 2), lambda i: (i, 0)),
         pl.BlockSpec((bn, 2), lambda i: (i, 0))),
    )


def _fin(x, w1, b1, w2, b2, bn=1000):
    n, k = x.shape
    f = w2.shape[1]
    assert n % bn == 0
    out_shape, out_specs = _proj_out_specs(n, bn)
    return pl.pallas_call(
        _fin_body,
        out_shape=out_shape,
        grid=(n // bn,),
        in_specs=[
            pl.BlockSpec((bn, k), lambda i: (i, 0)),
            pl.BlockSpec((k, HID), lambda i: (0, 0)),
            pl.BlockSpec((1, HID), lambda i: (0, 0)),
            pl.BlockSpec((HID, f), lambda i: (0, 0)),
            pl.BlockSpec((1, f), lambda i: (0, 0)),
        ],
        out_specs=out_specs,
    )(x, w1, b1.reshape(1, HID), w2, b2.reshape(1, f))


def _norm_specs(bn, f):
    return [
        pl.BlockSpec((bn, HID), lambda i: (i, 0)),
        pl.BlockSpec((bn, H), lambda i: (i, 0)),
        pl.BlockSpec((1, HID), lambda i: (0, 0)),
        pl.BlockSpec((1, HID), lambda i: (0, 0)),
        pl.BlockSpec((HID, f), lambda i: (0, 0)),
        pl.BlockSpec((1, f), lambda i: (0, 0)),
    ]


def _fmid(msg, den, g, b, w2, b2, bn):
    n = msg.shape[0]
    f = w2.shape[1]
    assert n % bn == 0
    out_shape, out_specs = _proj_out_specs(n, bn)
    return pl.pallas_call(
        _fmid_body,
        out_shape=out_shape,
        grid=(n // bn,),
        in_specs=_norm_specs(bn, f),
        out_specs=out_specs,
    )(msg, den, g.reshape(1, HID), b.reshape(1, HID), w2, b2.reshape(1, f))


def _fout(msg, den, g, b, w2, b2, bn):
    n = msg.shape[0]
    f = w2.shape[1]
    assert n % bn == 0
    return pl.pallas_call(
        _fout_body,
        out_shape=jax.ShapeDtypeStruct((n, f), jnp.float32),
        grid=(n // bn,),
        in_specs=_norm_specs(bn, f),
        out_specs=pl.BlockSpec((bn, f), lambda i: (i, 0)),
    )(msg, den, g.reshape(1, HID), b.reshape(1, HID), w2, b2.reshape(1, f))


def _sc_conv_body(ns, ndp, rpt, cw,
                  zflat, alsrc, aldst, srce, dste, zrows0, zden0,
                  msg_out, den_out,
                  eb_s, eb_d, gidxb, gdstb, alsb, aldb, exc,
                  zrow, msgb, bounce, denb, semi, semg, semz, accum, dena):
    c = lax.axis_index("c")
    s = lax.axis_index("s")
    nchunk = CHUNKS_PER_TILE
    tbase = s * (nchunk * IC)
    cns = c * ns
    cnd = c * ndp

    def issue_idx(i, b):
        off = tbase + jnp.minimum(i, nchunk - 1) * IC
        pltpu.async_copy(srce.at[pl.ds(off, IC)], eb_s.at[b], semi.at[b])
        pltpu.async_copy(dste.at[pl.ds(off, 128)], eb_d.at[b, 0], semi.at[b])
        pltpu.async_copy(dste.at[pl.ds(off + 128, 128)], eb_d.at[b, 1],
                         semi.at[b])

    def wait_idx(b):
        pltpu.make_async_copy(srce.at[pl.ds(0, IC)], eb_s.at[b],
                              semi.at[b]).wait()
        for j in range(2):
            pltpu.make_async_copy(dste.at[pl.ds(0, 128)], eb_d.at[b, j],
                                  semi.at[b]).wait()

    def build(b):
        for j in range(2):
            for h in range(8):
                sv = eb_s[b, pl.ds(j * 128 + h * 16, 16)]
                dv = eb_d[b, j, pl.ds(h * 16, 16)]
                gidxb[b, j, pl.ds(h * 16, 16)] = sv + cns
                gdstb[b, j, pl.ds(h * 16, 16)] = dv + cnd

    def issue_gathers(b):
        for j in range(2):
            pltpu.async_copy(alsrc.at[gidxb.at[b, j]],
                             alsb.at[b, pl.ds(j * 128, 128)], semg.at[b])
            pltpu.async_copy(aldst.at[gdstb.at[b, j]],
                             aldb.at[b, pl.ds(j * 128, 128)], semg.at[b])
            pltpu.async_copy(zflat.at[gidxb.at[b, j]],
                             zrow.at[b, pl.ds(j * 128, 128)], semz.at[b])

    def wait_al(b):
        for j in range(2):
            pltpu.make_async_copy(alsrc.at[gidxb.at[b, j]],
                                  alsb.at[b, pl.ds(j * 128, 128)],
                                  semg.at[b]).wait()
            pltpu.make_async_copy(aldst.at[gdstb.at[b, j]],
                                  aldb.at[b, pl.ds(j * 128, 128)],
                                  semg.at[b]).wait()

    def wait_z(b):
        for j in range(2):
            pltpu.make_async_copy(zflat.at[gidxb.at[b, j]],
                                  zrow.at[b, pl.ds(j * 128, 128)],
                                  semz.at[b]).wait()

    def compute_scatter(b):
        wait_al(b)
        exvals = []
        for g in range(IC // 16):
            av = alsb[b, pl.ds(g * 16, 16)] + aldb[b, pl.ds(g * 16, 16)]
            av = jnp.where(av >= 0, av, av * 0.2)
            ex = jnp.exp(av)
            exvals.append(ex)
            exc[pl.ds(g * 16, 16)] = ex
        wait_z(b)
        for e in range(IC):
            exs = exvals[e // 16][e % 16]
            lo, hi = plsc.unpack(plsc.bitcast(zrow[b, e, :], jnp.bfloat16),
                                 format=plsc.PackFormat.INTERLEAVED)
            msgb[e, pl.ds(0, 16)] = lo * exs
            msgb[e, pl.ds(16, 16)] = hi * exs
        for j in range(2):
            pltpu.sync_copy(msgb.at[pl.ds(j * 128, 128)],
                            accum.at[eb_d.at[b, j]], add=True)
            pltpu.sync_copy(exc.at[pl.ds(j * 128, 128)],
                            dena.at[eb_d.at[b, j]], add=True)

    issue_idx(0, 0)
    issue_idx(1, 1)

    pltpu.sync_copy(zrows0, bounce)
    pltpu.sync_copy(zden0, denb)
    base = s * rpt
    for k in range(rpt // cw):
        pltpu.sync_copy(bounce, accum.at[pl.ds(base + k * cw, cw)])
    pltpu.sync_copy(denb, dena.at[pl.ds(base, rpt)])
    plsc.subcore_barrier()

    wait_idx(0)
    build(0)
    issue_gathers(0)

    def body(kk, carry):
        t = kk * 2
        for b in range(2):
            i = t + b
            nb = (b + 1) % 2
            wait_idx(nb)
            build(nb)
            issue_gathers(nb)
            compute_scatter(b)
            issue_idx(i + 2, b)
        return carry

    lax.fori_loop(0, nchunk // 2, body, 0)

    wait_al(nchunk % 2)
    wait_z(nchunk % 2)
    wait_idx((nchunk + 1) % 2)

    plsc.subcore_barrier()

    for k in range(rpt // cw):
        r = base + k * cw
        pltpu.sync_copy(accum.at[pl.ds(r, cw)], bounce)
        pltpu.sync_copy(bounce, msg_out.at[pl.ds(r, cw), c])
    pltpu.sync_copy(dena.at[pl.ds(base, rpt)], denb)
    pltpu.sync_copy(denb, den_out.at[pl.ds(c * ndp + base, rpt)])


def _sc_conv(zflat, alsrc, aldst_p, src_p, dst_p, ns, ndp):
    rpt, cw = _row_split(ndp)
    mesh = plsc.VectorSubcoreMesh(core_axis_name="c", subcore_axis_name="s",
                                  num_cores=2, num_subcores=N_TILES)
    fn = pl.kernel(
        functools.partial(_sc_conv_body, ns, ndp, rpt, cw),
        out_type=(
            jax.ShapeDtypeStruct((ndp, 2, D), jnp.float32),
            jax.ShapeDtypeStruct((2 * ndp,), jnp.float32),
        ),
        mesh=mesh,
        compiler_params=pltpu.CompilerParams(needs_layout_passes=False,
                                             use_tc_tiling_on_sc=False),
        scratch_types=[
            pltpu.VMEM((2, IC), jnp.int32),
            pltpu.VMEM((2, 2, 128), jnp.int32),
            pltpu.VMEM((2, 2, 128), jnp.int32),
            pltpu.VMEM((2, 2, 128), jnp.int32),
            pltpu.VMEM((2, IC), jnp.float32),
            pltpu.VMEM((2, IC), jnp.float32),
            pltpu.VMEM((IC,), jnp.float32),
            pltpu.VMEM((2, IC, 16), jnp.uint32),
            pltpu.VMEM((IC, D), jnp.float32),
            pltpu.VMEM((cw, D), jnp.float32),
            pltpu.VMEM((rpt,), jnp.float32),
            pltpu.SemaphoreType.DMA((2,)),
            pltpu.SemaphoreType.DMA((2,)),
            pltpu.SemaphoreType.DMA((2,)),
            pltpu.VMEM_SHARED((ndp, D), jnp.float32),
            pltpu.VMEM_SHARED((ndp,), jnp.float32),
        ],
    )
    zrows0 = jnp.zeros((cw, D), jnp.float32)
    zden0 = jnp.zeros((rpt,), jnp.float32)
    msg, den = fn(zflat, alsrc, aldst_p, src_p, dst_p, zrows0, zden0)
    return msg.reshape(ndp, HID), den.reshape(2, ndp).T


def _block_attn_mat(a):
    z = jnp.zeros((D, 1), jnp.float32)
    return jnp.block([[a[0][:, None], z], [z, a[1][:, None]]])


def _pad_al(al, ndp):
    n = al.shape[0]
    return jnp.concatenate(
        [al, jnp.zeros((ndp - n, 2), jnp.float32)], axis=0).T.reshape(-1)


def _hm(al):
    return al.T.reshape(-1)


def _zhm(z, n):
    zb = z.astype(jnp.bfloat16).reshape(n, H, 2, 16)
    zt = zb.transpose(1, 0, 3, 2)
    zu = jax.lax.bitcast_convert_type(zt, jnp.uint32)
    return zu.reshape(H * n, 16)


def _layer_weights(p, l):
    a_src_d2s = _block_attn_mat(p[f"l{l}_asrc_d2s"])
    a_dst_d2s = _block_attn_mat(p[f"l{l}_adst_d2s"])
    a_src_s2d = _block_attn_mat(p[f"l{l}_asrc_s2d"])
    a_dst_s2d = _block_attn_mat(p[f"l{l}_adst_s2d"])
    w_d, b_d = p[f"l{l}_proj_diag_W"], p[f"l{l}_proj_diag_b"]
    w_s, b_s = p[f"l{l}_proj_stay_W"], p[f"l{l}_proj_stay_b"]
    wcat_d = jnp.concatenate([w_d, w_d @ a_src_d2s, w_d @ a_dst_s2d], axis=1)
    bcat_d = jnp.concatenate([b_d, b_d @ a_src_d2s, b_d @ a_dst_s2d])
    wcat_s = jnp.concatenate([w_s, w_s @ a_src_s2d, w_s @ a_dst_d2s], axis=1)
    bcat_s = jnp.concatenate([b_s, b_s @ a_src_s2d, b_s @ a_dst_d2s])
    return wcat_d, bcat_d, wcat_s, bcat_s


def kernel(x_stay, x_diag, params, ei_d2s_src, ei_d2s_dst, ei_s2d_src, ei_s2d_dst):
    p = params
    pad = EP - E
    e1s = jnp.concatenate([ei_d2s_src, jnp.zeros((pad,), jnp.int32)])
    e1d = jnp.concatenate([ei_d2s_dst, jnp.full((pad,), N_STAY, jnp.int32)])
    e2s = jnp.concatenate([ei_s2d_src, jnp.zeros((pad,), jnp.int32)])
    e2d = jnp.concatenate([ei_s2d_dst, jnp.full((pad,), N_DIAG, jnp.int32)])

    wcat_d0, bcat_d0, wcat_s0, bcat_s0 = _layer_weights(p, 0)
    z_diag, alsrc_d2s, aldst_s2d = _fin(
        x_diag, p["in_diag_W"], p["in_diag_b"], wcat_d0, bcat_d0)
    z_stay, alsrc_s2d, aldst_d2s = _fin(
        x_stay, p["in_stay_W"], p["in_stay_b"], wcat_s0, bcat_s0)

    msg_s, den_s = _sc_conv(_zhm(z_diag, N_DIAG), _hm(alsrc_d2s),
                            _pad_al(aldst_d2s, NDP_STAY),
                            e1s, e1d, N_DIAG, NDP_STAY)
    msg_d, den_d = _sc_conv(_zhm(z_stay, N_STAY), _hm(alsrc_s2d),
                            _pad_al(aldst_s2d, NDP_DIAG),
                            e2s, e2d, N_STAY, NDP_DIAG)

    wcat_d1, bcat_d1, wcat_s1, bcat_s1 = _layer_weights(p, 1)
    z_diag1, alsrc_d2s1, _ = _fmid(
        msg_d, den_d, p["l0_ln_g"], p["l0_ln_b"], wcat_d1, bcat_d1, bn=1024)
    z_stay1, _, aldst_d2s1 = _fmid(
        msg_s, den_s, p["l0_ln_g"], p["l0_ln_b"], wcat_s1, bcat_s1, bn=1088)

    msg_s1, den_s1 = _sc_conv(_zhm(z_diag1, NDP_DIAG), _hm(alsrc_d2s1),
                              _hm(aldst_d2s1),
                              e1s, e1d, NDP_DIAG, NDP_STAY)

    out = _fout(msg_s1, den_s1, p["l1_ln_g"], p["l1_ln_b"],
                p["clf_W"], p["clf_b"], bn=1088)
    return out[:N_STAY]
